# baseline (speedup 1.0000x reference)
import jax
import jax.numpy as jnp
from jax import lax
from jax.experimental import pallas as pl
from jax.experimental.pallas import tpu as pltpu
from jax.experimental.pallas import tpu_sc as plsc

_N = 10000
_E = 320000
_F = 128
_H = 16
_C = 2
_G = 64

_NC = 2
_NS = 16
_NW = _NC * _NS
_CH = 128
_K = 80
_EPT = _K * _CH
_EPAD = _NW * _EPT
_GRP = 8
_NG = _K // _GRP
_GCH = _GRP * _CH
_DD = 8
_ND = 128
_NACC = _N + _ND
_RPT = 624
_TAIL0 = _RPT * _NS
_TAIL = _N - _TAIL0

_mesh = plsc.VectorSubcoreMesh(core_axis_name="c", subcore_axis_name="s")


def _copy_rows(sid, src_at, dst_at):
  r0 = sid * _RPT
  pltpu.sync_copy(src_at(r0, _RPT), dst_at(r0, _RPT))

  @pl.when(sid == _NS - 1)
  def _():
    pltpu.sync_copy(src_at(_TAIL0, _TAIL), dst_at(_TAIL0, _TAIL))


def _zero_rows(sid, zeros_hbm, acc_sh):
  r0 = sid * _RPT
  for off, sz in ((0, 128), (128, 128), (256, 128), (384, 128), (512, 112)):
    pltpu.sync_copy(zeros_hbm.at[pl.ds(0, sz)], acc_sh.at[pl.ds(r0 + off, sz)])

  @pl.when(sid == _NS - 1)
  def _():
    pltpu.sync_copy(zeros_hbm.at[pl.ds(0, _TAIL)],
                    acc_sh.at[pl.ds(_TAIL0, _TAIL)])


def _seg_body_gather(y_hbm, zeros_hbm, src_hbm, dst_hbm, out_hbm,
                     src_v, dst_v, rows_v, acc_sh, tab_sh,
                     g0, g1, t0, t1):
  cid = lax.axis_index("c")
  sid = lax.axis_index("s")
  wid = sid * _NC + cid
  gsems = (g0, g1)
  ssems = (t0, t1)

  pltpu.sync_copy(src_hbm.at[wid], src_v)
  pltpu.sync_copy(dst_hbm.at[wid], dst_v)

  _copy_rows(sid, lambda o, n: y_hbm.at[pl.ds(o, n)],
             lambda o, n: tab_sh.at[pl.ds(o, n)])

  @pl.when(cid == 0)
  def _():
    _copy_rows(sid, lambda o, n: y_hbm.at[pl.ds(o, n)],
               lambda o, n: acc_sh.at[pl.ds(o, n)])

  @pl.when(cid != 0)
  def _():
    _zero_rows(sid, zeros_hbm, acc_sh)

  plsc.subcore_barrier()

  def _gather_group(g, p):
    for b in range(_GRP):
      pltpu.async_copy(tab_sh.at[src_v.at[g * _GRP + b]],
                       rows_v.at[p, pl.ds(b * _CH, _CH)], gsems[p])

  def _drain_gathers(p):
    pltpu.make_async_copy(y_hbm.at[pl.ds(0, _GCH)], rows_v.at[p],
                          gsems[p]).wait()

  def _scatter_group(g, p):
    for b in range(_GRP):
      pltpu.async_copy(rows_v.at[p, pl.ds(b * _CH, _CH)],
                       acc_sh.at[dst_v.at[g * _GRP + b]], ssems[p], add=True)

  def _drain_scatters(g, p):
    for b in range(_GRP):
      pltpu.make_async_copy(rows_v.at[p, pl.ds(b * _CH, _CH)],
                            acc_sh.at[dst_v.at[g * _GRP + b]],
                            ssems[p]).wait()

  def _run_group(g, p, prefetch):
    _drain_gathers(p)
    _scatter_group(g, p)
    _drain_scatters(g, p)
    if prefetch:
      _gather_group(g + 2, p)

  _gather_group(0, 0)
  _gather_group(1, 1)

  def pair(i, carry):
    _run_group(2 * i, 0, True)
    _run_group(2 * i + 1, 1, True)
    return carry

  lax.fori_loop(0, (_NG - 2) // 2, pair, 0)
  _run_group(_NG - 2, 0, False)
  _run_group(_NG - 1, 1, False)

  plsc.subcore_barrier()
  _copy_rows(sid, lambda o, n: acc_sh.at[pl.ds(o, n)],
             lambda o, n: out_hbm.at[cid, pl.ds(o, n)])


def _seg_body_ones(ones_hbm, zeros_hbm, dst_hbm, out_hbm,
                   dst_v, rows_v, acc_sh, s0, s1):
  cid = lax.axis_index("c")
  sid = lax.axis_index("s")
  wid = sid * _NC + cid
  sems = (s0, s1)

  pltpu.sync_copy(dst_hbm.at[wid], dst_v)
  _zero_rows(sid, zeros_hbm, acc_sh)
  pltpu.sync_copy(ones_hbm, rows_v)
  plsc.subcore_barrier()

  def _scat_group(g, p):
    for b in range(_GRP):
      pltpu.async_copy(rows_v, acc_sh.at[dst_v.at[g * _GRP + b]],
                       sems[p], add=True)

  def _drain_group(g, p):
    for b in range(_GRP):
      pltpu.make_async_copy(rows_v, acc_sh.at[dst_v.at[g * _GRP + b]],
                            sems[p]).wait()

  _scat_group(0, 0)
  _scat_group(1, 1)

  def pair(i, carry):
    for p in (0, 1):
      _drain_group(2 * i + p, p)
      _scat_group(2 * i + 2 + p, p)
    return carry

  lax.fori_loop(0, (_NG - 2) // 2, pair, 0)
  _drain_group(_NG - 2, 0)
  _drain_group(_NG - 1, 1)

  plsc.subcore_barrier()
  _copy_rows(sid, lambda o, n: acc_sh.at[pl.ds(o, n)],
             lambda o, n: out_hbm.at[cid, pl.ds(o, n)])


_seg_sum = pl.kernel(
    _seg_body_gather,
    out_type=jax.ShapeDtypeStruct((_NC, _N, _H), jnp.float32),
    mesh=_mesh,
    scratch_types=[
        pltpu.VMEM((_K, _CH), jnp.int32),
        pltpu.VMEM((_K, _CH), jnp.int32),
        pltpu.VMEM((2, _GCH, _H), jnp.float32),
        pltpu.VMEM_SHARED((_NACC, _H), jnp.float32),
        pltpu.VMEM_SHARED((_N, _H), jnp.float32),
    ] + [pltpu.SemaphoreType.DMA] * 4,
    name="gcn_seg_sum",
    compiler_params=pltpu.CompilerParams(use_tc_tiling_on_sc=False),
)

_deg_sum = pl.kernel(
    _seg_body_ones,
    out_type=jax.ShapeDtypeStruct((_NC, _N, _DD), jnp.float32),
    mesh=_mesh,
    scratch_types=[
        pltpu.VMEM((_K, _CH), jnp.int32),
        pltpu.VMEM((_CH, _DD), jnp.float32),
        pltpu.VMEM_SHARED((_NACC, _DD), jnp.float32),
    ] + [pltpu.SemaphoreType.DMA] * 2,
    name="gcn_deg",
    compiler_params=pltpu.CompilerParams(use_tc_tiling_on_sc=False),
)


def _tc0_body(x_ref, w1_ref, xw_ref):
  xw_ref[...] = jnp.dot(x_ref[...], w1_ref[...],
                        preferred_element_type=jnp.float32)


def _tc1_body(xw_ref, degp_ref, y_ref, dinv_ref):
  deg = degp_ref[0, :, 0:1] + degp_ref[1, :, 0:1] + 1.0
  dinv = lax.rsqrt(deg)
  y_ref[...] = xw_ref[...] * dinv
  dinv_ref[...] = dinv


def _tc2_body(p_ref, dinv_ref, b1_ref, w2_ref, y_ref):
  dinv = dinv_ref[...]
  h1 = jnp.maximum((p_ref[0] + p_ref[1]) * dinv + b1_ref[...], 0.0)
  t = jnp.dot(h1, w2_ref[...], preferred_element_type=jnp.float32)
  y_ref[...] = t * dinv


def _tc3_body(p_ref, dinv_ref, b2_ref, batch_ref, wf_ref, bf_ref, out_ref):
  dinv = dinv_ref[...]
  h2 = jnp.maximum((p_ref[0] + p_ref[1]) * dinv + b2_ref[...], 0.0)
  gids = lax.broadcasted_iota(jnp.int32, (_G, _N), 0)
  mask = (batch_ref[...] == gids).astype(jnp.float32)
  summed = jnp.dot(mask, h2, preferred_element_type=jnp.float32)
  counts = jnp.sum(mask, axis=1, keepdims=True)
  pooled = summed / jnp.maximum(counts, 1.0)
  logits = jnp.dot(pooled, wf_ref[...],
                   preferred_element_type=jnp.float32) + bf_ref[...]
  m = jnp.max(logits, axis=1, keepdims=True)
  s = logits - m
  out_ref[...] = s - jnp.log(jnp.sum(jnp.exp(s), axis=1, keepdims=True))


_tc0 = pl.pallas_call(
    _tc0_body,
    out_shape=jax.ShapeDtypeStruct((_N, _H), jnp.float32),
    name="gcn_tc0",
)

_tc1 = pl.pallas_call(
    _tc1_body,
    out_shape=(
        jax.ShapeDtypeStruct((_N, _H), jnp.float32),
        jax.ShapeDtypeStruct((_N, 1), jnp.float32),
    ),
    name="gcn_tc1",
)

_tc2 = pl.pallas_call(
    _tc2_body,
    out_shape=jax.ShapeDtypeStruct((_N, _H), jnp.float32),
    name="gcn_tc2",
)

_tc3 = pl.pallas_call(
    _tc3_body,
    out_shape=jax.ShapeDtypeStruct((_G, _C), jnp.float32),
    name="gcn_tc3",
)


@jax.jit
def kernel(x, edge_index, batch, W1, b1, W2, b2, Wf, bf):
  src = edge_index[0]
  dst = edge_index[1]
  npad = _EPAD - _E
  pad = jnp.arange(npad, dtype=jnp.int32)
  src_p = jnp.concatenate([src, pad]).reshape(_NW, _K, _CH)
  dst_p = jnp.concatenate([dst, _N + (pad % _ND)]).reshape(_NW, _K, _CH)
  zeros = jnp.zeros((_CH, _H), jnp.float32)
  zeros1 = jnp.zeros((_CH, _DD), jnp.float32)
  ones1 = jnp.ones((_CH, _DD), jnp.float32)

  xw = _tc0(x, W1)
  degp = _deg_sum(ones1, zeros1, dst_p)
  y1, dinv = _tc1(xw, degp)
  p1 = _seg_sum(y1, zeros, src_p, dst_p)
  y2 = _tc2(p1, dinv, b1.reshape(1, _H), W2)
  p2 = _seg_sum(y2, zeros, src_p, dst_p)
  return _tc3(p2, dinv, b2.reshape(1, _H), batch.reshape(1, _N),
              Wf, bf.reshape(1, _C))


# speedup vs baseline: 1.0207x; 1.0207x over previous
import jax
import jax.numpy as jnp
from jax import lax
from jax.experimental import pallas as pl
from jax.experimental.pallas import tpu as pltpu
from jax.experimental.pallas import tpu_sc as plsc

_N = 10000
_E = 320000
_F = 128
_H = 16
_C = 2
_G = 64

_NC = 2
_NS = 16
_NW = _NC * _NS
_CH = 128
_K = 79
_EPT = _K * _CH
_EPAD = _NW * _EPT
_NB = 4
_DD = 8
_ND = 128
_NACC = _N + _ND
_RPT = 624
_TAIL0 = _RPT * _NS
_TAIL = _N - _TAIL0

_mesh = plsc.VectorSubcoreMesh(core_axis_name="c", subcore_axis_name="s")


def _copy_rows(sid, src_at, dst_at):
  r0 = sid * _RPT
  pltpu.sync_copy(src_at(r0, _RPT), dst_at(r0, _RPT))

  @pl.when(sid == _NS - 1)
  def _():
    pltpu.sync_copy(src_at(_TAIL0, _TAIL), dst_at(_TAIL0, _TAIL))


def _zero_rows(sid, zeros_hbm, acc_sh):
  r0 = sid * _RPT
  for off, sz in ((0, 128), (128, 128), (256, 128), (384, 128), (512, 112)):
    pltpu.sync_copy(zeros_hbm.at[pl.ds(0, sz)], acc_sh.at[pl.ds(r0 + off, sz)])

  @pl.when(sid == _NS - 1)
  def _():
    pltpu.sync_copy(zeros_hbm.at[pl.ds(0, _TAIL)],
                    acc_sh.at[pl.ds(_TAIL0, _TAIL)])


def _seg_body_gather(y_hbm, zeros_hbm, src_hbm, dst_hbm, out_hbm,
                     src_v, dst_v, rows_v, acc_sh, tab_sh,
                     g0, g1, g2, g3, t0, t1, t2, t3):
  cid = lax.axis_index("c")
  sid = lax.axis_index("s")
  wid = sid * _NC + cid
  gsems = (g0, g1, g2, g3)
  ssems = (t0, t1, t2, t3)

  pltpu.sync_copy(src_hbm.at[wid], src_v)
  pltpu.sync_copy(dst_hbm.at[wid], dst_v)

  _copy_rows(sid, lambda o, n: y_hbm.at[pl.ds(o, n)],
             lambda o, n: tab_sh.at[pl.ds(o, n)])

  @pl.when(cid == 0)
  def _():
    _copy_rows(sid, lambda o, n: y_hbm.at[pl.ds(o, n)],
               lambda o, n: acc_sh.at[pl.ds(o, n)])

  @pl.when(cid != 0)
  def _():
    _zero_rows(sid, zeros_hbm, acc_sh)

  plsc.subcore_barrier()

  def _gather(j, sel):
    pltpu.async_copy(tab_sh.at[src_v.at[j]], rows_v.at[sel % 8],
                     gsems[sel % 4])

  def _drain_g(j, sel):
    pltpu.make_async_copy(tab_sh.at[src_v.at[j]], rows_v.at[sel % 8],
                          gsems[sel % 4]).wait()

  def _scat(j, sel):
    pltpu.async_copy(rows_v.at[sel % 8], acc_sh.at[dst_v.at[j]],
                     ssems[sel % 4], add=True)

  def _wait_s(j, sel):
    pltpu.make_async_copy(rows_v.at[sel % 8], acc_sh.at[dst_v.at[j]],
                          ssems[sel % 4]).wait()

  for j in range(4):
    _gather(j, j)
  for j in range(4):
    _drain_g(j, j)
    _scat(j, j)
    _gather(j + 4, j + 4)

  def group(i, carry):
    j0 = i * 8 + 4
    for b in range(8):
      j = j0 + b
      _drain_g(j, 4 + b)
      _wait_s(j - 4, b)
      _scat(j, 4 + b)
      _gather(j + 4, b)
    return carry

  lax.fori_loop(0, 8, group, 0)

  for j in range(68, _K):
    _drain_g(j, j)
    _wait_s(j - 4, j - 4)
    _scat(j, j)
    if j + 4 < _K:
      _gather(j + 4, j + 4)
  for j in range(_K - 4, _K):
    _wait_s(j, j)

  plsc.subcore_barrier()
  _copy_rows(sid, lambda o, n: acc_sh.at[pl.ds(o, n)],
             lambda o, n: out_hbm.at[cid, pl.ds(o, n)])


def _seg_body_ones(ones_hbm, zeros_hbm, dst_hbm, out_hbm,
                   dst_v, rows_v, acc_sh,
                   s0, s1, s2, s3, s4, s5, s6, s7):
  cid = lax.axis_index("c")
  sid = lax.axis_index("s")
  wid = sid * _NC + cid
  sems = (s0, s1, s2, s3, s4, s5, s6, s7)

  pltpu.sync_copy(dst_hbm.at[wid], dst_v)
  _zero_rows(sid, zeros_hbm, acc_sh)
  pltpu.sync_copy(ones_hbm, rows_v)
  plsc.subcore_barrier()

  def _scat(j, b):
    pltpu.async_copy(rows_v, acc_sh.at[dst_v.at[j]], sems[b % 8], add=True)

  def _wait(j, b):
    pltpu.make_async_copy(rows_v, acc_sh.at[dst_v.at[j]], sems[b % 8]).wait()

  for j in range(8):
    _scat(j, j)

  def grp(i, carry):
    j0 = i * 8
    for b in range(8):
      j = j0 + b
      _wait(j - 8, b)
      _scat(j, b)
    return carry

  lax.fori_loop(1, 9, grp, 0)
  for j in range(72, _K):
    _wait(j - 8, j)
    _scat(j, j)
  for j in range(_K - 8, _K):
    _wait(j, j)

  plsc.subcore_barrier()
  _copy_rows(sid, lambda o, n: acc_sh.at[pl.ds(o, n)],
             lambda o, n: out_hbm.at[cid, pl.ds(o, n)])


_seg_sum = pl.kernel(
    _seg_body_gather,
    out_type=jax.ShapeDtypeStruct((_NC, _N, _H), jnp.float32),
    mesh=_mesh,
    scratch_types=[
        pltpu.VMEM((_K, _CH), jnp.int32),
        pltpu.VMEM((_K, _CH), jnp.int32),
        pltpu.VMEM((8, _CH, _H), jnp.float32),
        pltpu.VMEM_SHARED((_NACC, _H), jnp.float32),
        pltpu.VMEM_SHARED((_N, _H), jnp.float32),
    ] + [pltpu.SemaphoreType.DMA] * 8,
    name="gcn_seg_sum",
    compiler_params=pltpu.CompilerParams(use_tc_tiling_on_sc=False),
)

_deg_sum = pl.kernel(
    _seg_body_ones,
    out_type=jax.ShapeDtypeStruct((_NC, _N, _DD), jnp.float32),
    mesh=_mesh,
    scratch_types=[
        pltpu.VMEM((_K, _CH), jnp.int32),
        pltpu.VMEM((_CH, _DD), jnp.float32),
        pltpu.VMEM_SHARED((_NACC, _DD), jnp.float32),
    ] + [pltpu.SemaphoreType.DMA] * 8,
    name="gcn_deg",
    compiler_params=pltpu.CompilerParams(use_tc_tiling_on_sc=False),
)


def _tc0_body(x_ref, w1_ref, xw_ref):
  xw_ref[...] = jnp.dot(x_ref[...], w1_ref[...],
                        preferred_element_type=jnp.float32)


def _tc1_body(xw_ref, degp_ref, y_ref, dinv_ref):
  deg = degp_ref[0, :, 0:1] + degp_ref[1, :, 0:1] + 1.0
  dinv = lax.rsqrt(deg)
  y_ref[...] = xw_ref[...] * dinv
  dinv_ref[...] = dinv


def _tc2_body(p_ref, dinv_ref, b1_ref, w2_ref, y_ref):
  dinv = dinv_ref[...]
  h1 = jnp.maximum((p_ref[0] + p_ref[1]) * dinv + b1_ref[...], 0.0)
  t = jnp.dot(h1, w2_ref[...], preferred_element_type=jnp.float32)
  y_ref[...] = t * dinv


def _tc3_body(p_ref, dinv_ref, b2_ref, batch_ref, wf_ref, bf_ref, out_ref):
  dinv = dinv_ref[...]
  h2 = jnp.maximum((p_ref[0] + p_ref[1]) * dinv + b2_ref[...], 0.0)
  gids = lax.broadcasted_iota(jnp.int32, (_G, _N), 0)
  mask = (batch_ref[...] == gids).astype(jnp.float32)
  summed = jnp.dot(mask, h2, preferred_element_type=jnp.float32)
  counts = jnp.sum(mask, axis=1, keepdims=True)
  pooled = summed / jnp.maximum(counts, 1.0)
  logits = jnp.dot(pooled, wf_ref[...],
                   preferred_element_type=jnp.float32) + bf_ref[...]
  m = jnp.max(logits, axis=1, keepdims=True)
  s = logits - m
  out_ref[...] = s - jnp.log(jnp.sum(jnp.exp(s), axis=1, keepdims=True))


_tc0 = pl.pallas_call(
    _tc0_body,
    out_shape=jax.ShapeDtypeStruct((_N, _H), jnp.float32),
    name="gcn_tc0",
)

_tc1 = pl.pallas_call(
    _tc1_body,
    out_shape=(
        jax.ShapeDtypeStruct((_N, _H), jnp.float32),
        jax.ShapeDtypeStruct((_N, 1), jnp.float32),
    ),
    name="gcn_tc1",
)

_tc2 = pl.pallas_call(
    _tc2_body,
    out_shape=jax.ShapeDtypeStruct((_N, _H), jnp.float32),
    name="gcn_tc2",
)

_tc3 = pl.pallas_call(
    _tc3_body,
    out_shape=jax.ShapeDtypeStruct((_G, _C), jnp.float32),
    name="gcn_tc3",
)


@jax.jit
def kernel(x, edge_index, batch, W1, b1, W2, b2, Wf, bf):
  src = edge_index[0]
  dst = edge_index[1]
  npad = _EPAD - _E
  pad = jnp.arange(npad, dtype=jnp.int32)
  src_p = jnp.concatenate([src, pad]).reshape(_NW, _K, _CH)
  dst_p = jnp.concatenate([dst, _N + (pad % _ND)]).reshape(_NW, _K, _CH)
  zeros = jnp.zeros((_CH, _H), jnp.float32)
  zeros1 = jnp.zeros((_CH, _DD), jnp.float32)
  ones1 = jnp.ones((_CH, _DD), jnp.float32)

  xw = _tc0(x, W1)
  degp = _deg_sum(ones1, zeros1, dst_p)
  y1, dinv = _tc1(xw, degp)
  p1 = _seg_sum(y1, zeros, src_p, dst_p)
  y2 = _tc2(p1, dinv, b1.reshape(1, _H), W2)
  p2 = _seg_sum(y2, zeros, src_p, dst_p)
  return _tc3(p2, dinv, b2.reshape(1, _H), batch.reshape(1, _N),
              Wf, bf.reshape(1, _C))
